# Initial kernel scaffold; baseline (speedup 1.0000x reference)
import jax
import jax.numpy as jnp
from jax.experimental import pallas as pl


def kernel(fluidPositions, boundaryPositions, fluidFeatures, boundaryFeatures, support, W0, b0, W1, b1, W2, b2, W3, b3, fcW0, fcb0, fcW1, fcb1, fcW2, fcb2, fcW3, fcb3):
    raise NotImplementedError("write your pallas kernel here")


# speedup vs baseline: 2.4569x; 2.4569x over previous
import functools

import jax
import jax.numpy as jnp
import numpy as np
from jax.experimental import pallas as pl

N_NODES = 10000
NP = 10240
BLK = 256
CC = 512
NB = 4
MB = 4
P = NB * MB
_INV_PI = float(1.0 / np.pi)


def _mm_body(a_ref, b_ref, bias_ref, *rest, act, res):
    if res:
        r_ref, o_ref = rest
    else:
        (o_ref,) = rest
    a = a_ref[...]
    if act:
        a = jnp.maximum(a, 0.0)
    o = jnp.dot(a, b_ref[...], preferred_element_type=jnp.float32) + bias_ref[...]
    if res:
        o = o + r_ref[...]
    o_ref[...] = o


def _mm(a, b, bias, act=False, res=None):
    m, k = a.shape
    _, n = b.shape
    inputs = [a, b, bias.reshape(1, n)]
    specs = [
        pl.BlockSpec((CC, k), lambda i: (i, 0)),
        pl.BlockSpec((k, n), lambda i: (0, 0)),
        pl.BlockSpec((1, n), lambda i: (0, 0)),
    ]
    if res is not None:
        inputs.append(res)
        specs.append(pl.BlockSpec((CC, n), lambda i: (i, 0)))
    return pl.pallas_call(
        functools.partial(_mm_body, act=act, res=res is not None),
        grid=(m // CC,),
        in_specs=specs,
        out_specs=pl.BlockSpec((CC, n), lambda i: (i, 0)),
        out_shape=jax.ShapeDtypeStruct((m, n), jnp.float32),
    )(*inputs)


def _hat(x, center):
    return jnp.maximum(0.0, 1.0 - jnp.abs(x - center) * 1.5)


def _conv_body(s_ref, q_ref, c_ref, y_ref, b_ref, a_ref, o_ref, *, nco, scale):
    i = pl.program_id(0)
    j = pl.program_id(1)
    ncols = pl.num_programs(1)
    s = s_ref[0, 0]
    q = q_ref[...]
    c = c_ref[...]
    dx = q[:, 0:1] - c[:, 0][None, :]
    dy = q[:, 1:2] - c[:, 1][None, :]
    d2 = dx * dx + dy * dy
    rows = i * BLK + jax.lax.broadcasted_iota(jnp.int32, (BLK, CC), 0)
    cols = j * CC + jax.lax.broadcasted_iota(jnp.int32, (BLK, CC), 1)
    mask = (d2 < s * s) & (rows != cols)
    inv = 1.0 / s
    ex = jnp.clip(dx * inv, -1.0, 1.0)
    ey = jnp.clip(dy * inv, -1.0, 1.0)
    r = jnp.sqrt(ex * ex + ey * ey)
    th = jnp.arctan2(ey, ex) * _INV_PI
    ru = 2.0 * r - 1.0
    bu = [_hat(ru, -1.0), _hat(ru, -1.0 / 3.0), _hat(ru, 1.0 / 3.0), _hat(ru, 1.0)]
    bv = [_hat(th, -1.0), _hat(th, -1.0 / 3.0), _hat(th, 1.0 / 3.0), _hat(th, 1.0)]
    y = y_ref[...]
    acc = jnp.zeros((BLK, nco), jnp.float32)
    for u in range(NB):
        for v in range(MB):
            p = u * MB + v
            A = jnp.where(mask, bu[u] * bv[v], 0.0)
            acc += jnp.dot(A, y[:, p * nco:(p + 1) * nco],
                           preferred_element_type=jnp.float32)

    @pl.when(j == 0)
    def _():
        o_ref[...] = jnp.zeros_like(o_ref)

    o_ref[...] += acc

    @pl.when(j == ncols - 1)
    def _():
        o_ref[...] = (o_ref[...] + b_ref[...] + a_ref[...]) * scale


def _conv(s, pos, y, b, addend, scale=1.0):
    nco = b.shape[0]
    return pl.pallas_call(
        functools.partial(_conv_body, nco=nco, scale=scale),
        grid=(NP // BLK, NP // CC),
        in_specs=[
            pl.BlockSpec((1, 1), lambda i, j: (0, 0)),
            pl.BlockSpec((BLK, 2), lambda i, j: (i, 0)),
            pl.BlockSpec((CC, 2), lambda i, j: (j, 0)),
            pl.BlockSpec((CC, P * nco), lambda i, j: (j, 0)),
            pl.BlockSpec((1, nco), lambda i, j: (0, 0)),
            pl.BlockSpec((BLK, nco), lambda i, j: (i, 0)),
        ],
        out_specs=pl.BlockSpec((BLK, nco), lambda i, j: (i, 0)),
        out_shape=jax.ShapeDtypeStruct((NP, nco), jnp.float32),
    )(s, pos, pos, y, b.reshape(1, nco), addend)


def _wt(W):
    nb, mb, cin, cout = W.shape
    return W.reshape(P, cin, cout).transpose(1, 0, 2).reshape(cin, P * cout)


def kernel(fluidPositions, boundaryPositions, fluidFeatures, boundaryFeatures,
           support, W0, b0, W1, b1, W2, b2, W3, b3, fcW0, fcb0, fcW1, fcb1,
           fcW2, fcb2, fcW3, fcb3):
    pad = NP - N_NODES
    pos = jnp.concatenate(
        [fluidPositions, jnp.full((pad, 2), 1e6, jnp.float32)], axis=0)
    x0 = jnp.concatenate(
        [fluidFeatures, jnp.zeros((pad, fluidFeatures.shape[1]), jnp.float32)],
        axis=0)
    s = jnp.reshape(support, (1, 1))
    z32 = jnp.zeros((512,), jnp.float32)

    y0 = _mm(x0, _wt(W0), z32)
    lin0 = _mm(x0, fcW0.T, fcb0)
    conv0 = _conv(s, pos, y0, b0, jnp.zeros((NP, 32), jnp.float32))
    ans0 = jnp.concatenate([lin0, conv0], axis=1)

    y1 = _mm(ans0, _wt(W1), z32, act=True)
    lin1 = _mm(ans0, fcW1.T, fcb1, act=True)
    ans1 = _conv(s, pos, y1, b1, lin1)

    y2 = _mm(ans1, _wt(W2), z32, act=True)
    lin2 = _mm(ans1, fcW2.T, fcb2, act=True, res=ans1)
    ans2 = _conv(s, pos, y2, b2, lin2)

    y3 = _mm(ans2, _wt(W3), z32[:P * 2], act=True)
    lin3 = _mm(ans2, fcW3.T, fcb3, act=True)
    out = _conv(s, pos, y3, b3, lin3, scale=1.0 / 128.0)
    return out[:N_NODES]
